# baseline (speedup 1.0000x reference)
import functools

import jax
import jax.numpy as jnp
from jax import lax
from jax.experimental import pallas as pl
from jax.experimental.pallas import tpu as pltpu
from jax.experimental.pallas import tpu_sc as plsc

BATCH = 16384
K = 128
LANES = 16
NC = 2
NS = 16
NW = NC * NS
ROWS_PER_W = BATCH // NW
CHUNK = 64
NCHUNK = ROWS_PER_W // CHUNK
NSLOT = 3
NSPREG = 2

_mesh = plsc.VectorSubcoreMesh(core_axis_name="c", subcore_axis_name="s")

_scratch = (
    [pltpu.VMEM((ROWS_PER_W,), jnp.int32)] * 2
    + [pltpu.VMEM((CHUNK, K), jnp.float32)] * (2 * NSLOT)
    + [pltpu.VMEM((ROWS_PER_W,), jnp.float32)]
    + [pltpu.VMEM_SHARED((NS, NSPREG * CHUNK, K), jnp.float32)] * 2
    + [pltpu.SemaphoreType.DMA] * (2 * NSLOT + NSPREG)
)


@functools.partial(
    pl.kernel,
    out_type=[
        jax.ShapeDtypeStruct((BATCH,), jnp.float32),
        jax.ShapeDtypeStruct((BATCH, K), jnp.float32),
        jax.ShapeDtypeStruct((BATCH, K), jnp.float32),
    ],
    mesh=_mesh,
    scratch_types=_scratch,
    compiler_params=pltpu.CompilerParams(needs_layout_passes=False),
)
def _sc_body(users_hbm, items_hbm, gu_hbm, gi_hbm,
             xui_hbm, gu_out_hbm, gi_out_hbm, *scr):
    idxu_v, idxi_v = scr[0], scr[1]
    ubufs = scr[2:2 + NSLOT]
    ibufs = scr[2 + NSLOT:2 + 2 * NSLOT]
    xui_v = scr[2 + 2 * NSLOT]
    shu, shi = scr[3 + 2 * NSLOT], scr[4 + 2 * NSLOT]
    sems = scr[5 + 2 * NSLOT:]
    sg = sems[0:NSLOT]
    sx = sems[NSLOT:2 * NSLOT]
    sh = sems[2 * NSLOT:2 * NSLOT + NSPREG]

    cid = lax.axis_index("c")
    sid = lax.axis_index("s")
    wid = sid * NC + cid
    base = wid * ROWS_PER_W

    pltpu.sync_copy(users_hbm.at[pl.ds(base, ROWS_PER_W)], idxu_v)
    pltpu.sync_copy(items_hbm.at[pl.ds(base, ROWS_PER_W)], idxi_v)

    lane = lax.iota(jnp.int32, LANES)

    def fire_gather(n, s):
        co = n * CHUNK
        cu = pltpu.async_copy(gu_hbm.at[idxu_v.at[pl.ds(co, CHUNK)]], ubufs[s], sg[s])
        ci = pltpu.async_copy(gi_hbm.at[idxi_v.at[pl.ds(co, CHUNK)]], ibufs[s], sg[s])
        return cu, ci

    def fire_xbar(s, q):
        xu = pltpu.async_copy(ubufs[s], shu.at[sid, pl.ds(q * CHUNK, CHUNK)], sx[s])
        xi = pltpu.async_copy(ibufs[s], shi.at[sid, pl.ds(q * CHUNK, CHUNK)], sx[s])
        return xu, xi

    def fire_hbm_write(n, q):
        co = n * CHUNK
        wu = pltpu.async_copy(shu.at[sid, pl.ds(q * CHUNK, CHUNK)],
                              gu_out_hbm.at[pl.ds(base + co, CHUNK)], sh[q])
        wi = pltpu.async_copy(shi.at[sid, pl.ds(q * CHUNK, CHUNK)],
                              gi_out_hbm.at[pl.ds(base + co, CHUNK)], sh[q])
        return wu, wi

    def dot_chunk(n, s):
        co = n * CHUNK
        u_buf, i_buf = ubufs[s], ibufs[s]

        def group_body(g, _):
            base_r = g * LANES

            def row_body(j, out16):
                ur = u_buf.at[base_r + j]
                ir = i_buf.at[base_r + j]
                acc = ur[pl.ds(0, LANES)] * ir[pl.ds(0, LANES)]
                for k in range(1, K // LANES):
                    acc = acc + ur[pl.ds(k * LANES, LANES)] * ir[pl.ds(k * LANES, LANES)]
                return out16 + jnp.where(lane == j, jnp.sum(acc), 0.0)

            out16 = lax.fori_loop(0, LANES, row_body,
                                  jnp.zeros((LANES,), jnp.float32), unroll=2)
            xui_v[pl.ds(co + base_r, LANES)] = out16
            return 0

        lax.fori_loop(0, CHUNK // LANES, group_body, 0)

    gath = {}
    xbar = {}
    hbmw = {}
    for n in range(NSLOT):
        gath[n] = fire_gather(n, n)

    for n in range(NCHUNK):
        s = n % NSLOT
        q = n % NSPREG
        for c in gath[n]:
            c.wait()
        if n >= NSPREG:
            for c in hbmw[n - NSPREG]:
                c.wait()
        xbar[n] = fire_xbar(s, q)
        dot_chunk(n, s)
        for c in xbar[n]:
            c.wait()
        hbmw[n] = fire_hbm_write(n, q)
        if n + NSLOT < NCHUNK:
            gath[n + NSLOT] = fire_gather(n + NSLOT, s)

    for n in range(max(0, NCHUNK - NSPREG), NCHUNK):
        for c in hbmw[n]:
            c.wait()

    pltpu.sync_copy(xui_v, xui_hbm.at[pl.ds(base, ROWS_PER_W)])


def kernel(users, items, Gu, Gi):
    xui, gu, gi = _sc_body(users.astype(jnp.int32), items.astype(jnp.int32), Gu, Gi)
    return xui, gu, gi


# speedup vs baseline: 1.0441x; 1.0441x over previous
import functools

import jax
import jax.numpy as jnp
from jax import lax
from jax.experimental import pallas as pl
from jax.experimental.pallas import tpu as pltpu
from jax.experimental.pallas import tpu_sc as plsc

BATCH = 16384
K = 128
LANES = 16
NC = 2
NS = 16
NW = NC * NS
ROWS_PER_W = BATCH // NW
CHUNK = 128
NCHUNK = ROWS_PER_W // CHUNK
NSLOT = 2

_mesh = plsc.VectorSubcoreMesh(core_axis_name="c", subcore_axis_name="s")

_scratch = (
    [pltpu.VMEM((ROWS_PER_W,), jnp.int32)] * 2
    + [pltpu.VMEM((CHUNK, K), jnp.float32)] * (2 * NSLOT)
    + [pltpu.VMEM((ROWS_PER_W,), jnp.float32)]
    + [pltpu.SemaphoreType.DMA] * (2 * NSLOT)
)


@functools.partial(
    pl.kernel,
    out_type=[
        jax.ShapeDtypeStruct((BATCH,), jnp.float32),
        jax.ShapeDtypeStruct((BATCH, K), jnp.float32),
        jax.ShapeDtypeStruct((BATCH, K), jnp.float32),
    ],
    mesh=_mesh,
    scratch_types=_scratch,
    compiler_params=pltpu.CompilerParams(needs_layout_passes=False),
)
def _sc_body(users_hbm, items_hbm, gu_hbm, gi_hbm,
             xui_hbm, gu_out_hbm, gi_out_hbm, *scr):
    idxu_v, idxi_v = scr[0], scr[1]
    ubufs = scr[2:2 + NSLOT]
    ibufs = scr[2 + NSLOT:2 + 2 * NSLOT]
    xui_v = scr[2 + 2 * NSLOT]
    sg = scr[3 + 2 * NSLOT:3 + 3 * NSLOT]
    sw = scr[3 + 3 * NSLOT:3 + 4 * NSLOT]

    wid = lax.axis_index("s") * NC + lax.axis_index("c")
    base = wid * ROWS_PER_W

    pltpu.sync_copy(users_hbm.at[pl.ds(base, ROWS_PER_W)], idxu_v)
    pltpu.sync_copy(items_hbm.at[pl.ds(base, ROWS_PER_W)], idxi_v)

    lane = lax.iota(jnp.int32, LANES)

    def fire_gather(n, s):
        co = n * CHUNK
        cu = pltpu.async_copy(gu_hbm.at[idxu_v.at[pl.ds(co, CHUNK)]], ubufs[s], sg[s])
        ci = pltpu.async_copy(gi_hbm.at[idxi_v.at[pl.ds(co, CHUNK)]], ibufs[s], sg[s])
        return cu, ci

    def fire_writeback(n, s):
        co = n * CHUNK
        wu = pltpu.async_copy(ubufs[s], gu_out_hbm.at[pl.ds(base + co, CHUNK)], sw[s])
        wi = pltpu.async_copy(ibufs[s], gi_out_hbm.at[pl.ds(base + co, CHUNK)], sw[s])
        return wu, wi

    def dot_chunk(n, s):
        co = n * CHUNK
        u_buf, i_buf = ubufs[s], ibufs[s]

        def group_body(g, _):
            base_r = g * LANES

            def row_body(j, out16):
                ur = u_buf.at[base_r + j]
                ir = i_buf.at[base_r + j]
                acc = ur[pl.ds(0, LANES)] * ir[pl.ds(0, LANES)]
                for k in range(1, K // LANES):
                    acc = acc + ur[pl.ds(k * LANES, LANES)] * ir[pl.ds(k * LANES, LANES)]
                return out16 + jnp.where(lane == j, jnp.sum(acc), 0.0)

            out16 = lax.fori_loop(0, LANES, row_body,
                                  jnp.zeros((LANES,), jnp.float32), unroll=2)
            xui_v[pl.ds(co + base_r, LANES)] = out16
            return 0

        lax.fori_loop(0, CHUNK // LANES, group_body, 0)

    gath = {}
    wbs = {}
    for n in range(NSLOT):
        gath[n] = fire_gather(n, n)

    for n in range(NCHUNK):
        s = n % NSLOT
        for c in gath[n]:
            c.wait()
        wbs[n] = fire_writeback(n, s)
        dot_chunk(n, s)
        if n + NSLOT < NCHUNK:
            for c in wbs[n]:
                c.wait()
            gath[n + NSLOT] = fire_gather(n + NSLOT, s)

    for n in range(max(0, NCHUNK - NSLOT), NCHUNK):
        for c in wbs[n]:
            c.wait()

    pltpu.sync_copy(xui_v, xui_hbm.at[pl.ds(base, ROWS_PER_W)])


def kernel(users, items, Gu, Gi):
    xui, gu, gi = _sc_body(users.astype(jnp.int32), items.astype(jnp.int32), Gu, Gi)
    return xui, gu, gi


# speedup vs baseline: 1.0741x; 1.0288x over previous
import functools

import jax
import jax.numpy as jnp
from jax import lax
from jax.experimental import pallas as pl
from jax.experimental.pallas import tpu as pltpu
from jax.experimental.pallas import tpu_sc as plsc

BATCH = 16384
K = 128
LANES = 16
NC = 2
NS = 16
NW = NC * NS
ROWS_PER_W = BATCH // NW
CHUNK = 128
NCHUNK = ROWS_PER_W // CHUNK
NSLOT = 3

_mesh = plsc.VectorSubcoreMesh(core_axis_name="c", subcore_axis_name="s")

_scratch = (
    [pltpu.VMEM((ROWS_PER_W,), jnp.int32)] * 2
    + [pltpu.VMEM((CHUNK, K), jnp.float32)] * (2 * NSLOT)
    + [pltpu.VMEM((ROWS_PER_W,), jnp.float32)]
    + [pltpu.SemaphoreType.DMA] * (2 * NSLOT)
)


@functools.partial(
    pl.kernel,
    out_type=[
        jax.ShapeDtypeStruct((BATCH,), jnp.float32),
        jax.ShapeDtypeStruct((BATCH, K), jnp.float32),
        jax.ShapeDtypeStruct((BATCH, K), jnp.float32),
    ],
    mesh=_mesh,
    scratch_types=_scratch,
    compiler_params=pltpu.CompilerParams(needs_layout_passes=False),
)
def _sc_body(users_hbm, items_hbm, gu_hbm, gi_hbm,
             xui_hbm, gu_out_hbm, gi_out_hbm, *scr):
    idxu_v, idxi_v = scr[0], scr[1]
    ubufs = scr[2:2 + NSLOT]
    ibufs = scr[2 + NSLOT:2 + 2 * NSLOT]
    xui_v = scr[2 + 2 * NSLOT]
    sg = scr[3 + 2 * NSLOT:3 + 3 * NSLOT]
    sw = scr[3 + 3 * NSLOT:3 + 4 * NSLOT]

    wid = lax.axis_index("s") * NC + lax.axis_index("c")
    base = wid * ROWS_PER_W

    pltpu.sync_copy(users_hbm.at[pl.ds(base, ROWS_PER_W)], idxu_v)
    pltpu.sync_copy(items_hbm.at[pl.ds(base, ROWS_PER_W)], idxi_v)

    lane = lax.iota(jnp.int32, LANES)

    def fire_gather(n, s):
        co = n * CHUNK
        cu = pltpu.async_copy(gu_hbm.at[idxu_v.at[pl.ds(co, CHUNK)]], ubufs[s], sg[s])
        ci = pltpu.async_copy(gi_hbm.at[idxi_v.at[pl.ds(co, CHUNK)]], ibufs[s], sg[s])
        return cu, ci

    def fire_writeback(n, s):
        co = n * CHUNK
        wu = pltpu.async_copy(ubufs[s], gu_out_hbm.at[pl.ds(base + co, CHUNK)], sw[s])
        wi = pltpu.async_copy(ibufs[s], gi_out_hbm.at[pl.ds(base + co, CHUNK)], sw[s])
        return wu, wi

    def dot_chunk(n, s):
        co = n * CHUNK
        u_buf, i_buf = ubufs[s], ibufs[s]

        def group_body(g, _):
            base_r = g * LANES

            def row_body(j, out16):
                ur = u_buf.at[base_r + j]
                ir = i_buf.at[base_r + j]
                acc = ur[pl.ds(0, LANES)] * ir[pl.ds(0, LANES)]
                for k in range(1, K // LANES):
                    acc = acc + ur[pl.ds(k * LANES, LANES)] * ir[pl.ds(k * LANES, LANES)]
                return out16 + jnp.where(lane == j, jnp.sum(acc), 0.0)

            out16 = lax.fori_loop(0, LANES, row_body,
                                  jnp.zeros((LANES,), jnp.float32), unroll=2)
            xui_v[pl.ds(co + base_r, LANES)] = out16
            return 0

        lax.fori_loop(0, CHUNK // LANES, group_body, 0)

    gath = {}
    wbs = {}
    for n in range(NSLOT):
        gath[n] = fire_gather(n, n)

    for n in range(NCHUNK):
        s = n % NSLOT
        for c in gath[n]:
            c.wait()
        wbs[n] = fire_writeback(n, s)
        dot_chunk(n, s)
        if n + NSLOT < NCHUNK:
            for c in wbs[n]:
                c.wait()
            gath[n + NSLOT] = fire_gather(n + NSLOT, s)

    pltpu.sync_copy(xui_v, xui_hbm.at[pl.ds(base, ROWS_PER_W)])

    for n in range(max(0, NCHUNK - NSLOT), NCHUNK):
        for c in wbs[n]:
            c.wait()


def kernel(users, items, Gu, Gi):
    xui, gu, gi = _sc_body(users.astype(jnp.int32), items.astype(jnp.int32), Gu, Gi)
    return xui, gu, gi
